# Initial kernel scaffold; baseline (speedup 1.0000x reference)
import functools

import jax
import jax.numpy as jnp
from jax import lax
from jax.experimental import pallas as pl
from jax.experimental.pallas import tpu as pltpu
from jax.experimental.pallas import tpu_sc as plsc

F = 64
FC = 16
P = F // FC
NC = 2
NS = 16
B = 128


def _spmm_body(n_dst_pad, g_per_tile, x_hbm, sidx_hbm, didx_hbm, val_hbm,
               out_hbm, sidx_v, didx_v, val_v, rows_v, zbuf_v, acc, gsem):
    c = lax.axis_index("c")
    s = lax.axis_index("s")
    w = c * NS + s
    zrows = n_dst_pad // NS

    def _zb(r, carry):
        zbuf_v[r] = jnp.zeros((FC,), jnp.float32)
        return carry

    lax.fori_loop(0, B, _zb, 0)

    for p in range(P):
        def _zero(i, carry):
            pltpu.sync_copy(zbuf_v, acc.at[pl.ds(s * zrows + i * B, B)])
            return carry

        lax.fori_loop(0, zrows // B, _zero, 0)
        plsc.subcore_barrier()

        def _batch(g, carry):
            row = w * g_per_tile + g
            pltpu.sync_copy(sidx_hbm.at[pl.ds(row, 1)], sidx_v)
            pltpu.sync_copy(didx_hbm.at[pl.ds(row, 1)], didx_v)
            pltpu.sync_copy(val_hbm.at[pl.ds(row, 1)], val_v)
            pltpu.async_copy(x_hbm.at[p].at[sidx_v.at[0]], rows_v, gsem).wait()

            def _scale(e, inner):
                rows_v[e] = rows_v[e] * val_v[0, e]
                return inner

            lax.fori_loop(0, B, _scale, 0)
            pltpu.sync_copy(rows_v, acc.at[didx_v.at[0]], add=True)
            return carry

        lax.fori_loop(0, g_per_tile, _batch, 0)
        plsc.subcore_barrier()

        pltpu.sync_copy(acc.at[pl.ds(s * zrows, zrows)],
                        out_hbm.at[p, c, pl.ds(s * zrows, zrows)])
        plsc.subcore_barrier()


def _make_spmm(n_dst_pad, g_per_tile):
    mesh = plsc.VectorSubcoreMesh(core_axis_name="c", subcore_axis_name="s",
                                  num_cores=NC, num_subcores=NS)
    return pl.kernel(
        functools.partial(_spmm_body, n_dst_pad, g_per_tile),
        out_type=jax.ShapeDtypeStruct((P, NC, n_dst_pad, FC), jnp.float32),
        mesh=mesh,
        scratch_types=[
            pltpu.VMEM((1, B), jnp.int32),
            pltpu.VMEM((1, B), jnp.int32),
            pltpu.VMEM((1, B), jnp.float32),
            pltpu.VMEM((B, FC), jnp.float32),
            pltpu.VMEM((B, FC), jnp.float32),
            pltpu.VMEM_SHARED((n_dst_pad, FC), jnp.float32),
            pltpu.SemaphoreType.DMA,
        ],
    )


def _ceil_to(x, m):
    return (x + m - 1) // m * m


def _split(x):
    n = x.shape[0]
    return x.reshape(n, P, FC).transpose(1, 0, 2)


def _unsplit(x4):
    n = x4.shape[1]
    return x4.transpose(1, 0, 2).reshape(n, F)


def kernel(embed_user, embed_item, u_idx, i_idx, ui_val, iu_val, adj_val, d_i, d_j):
    n_users, _ = embed_user.shape
    n_items = embed_item.shape[0]
    n_edges = u_idx.shape[0]

    e_pad = _ceil_to(n_edges, NC * NS * B)
    g_per_tile = e_pad // (NC * NS * B)
    up = _ceil_to(n_users, NS * B)
    ip = _ceil_to(n_items, NS * B)
    pad = e_pad - n_edges

    def prep_idx(idx):
        return jnp.pad(idx.astype(jnp.int32), (0, pad)).reshape(-1, B)

    def prep_val(v):
        return jnp.pad(v, (0, pad)).reshape(-1, B)

    src_u = prep_idx(i_idx)
    dst_u = prep_idx(u_idx)
    src_i = prep_idx(u_idx)
    dst_i = prep_idx(i_idx)
    v_ui = prep_val(ui_val)
    v_iu = prep_val(iu_val)
    v_adj = prep_val(adj_val)

    spmm_u = _make_spmm(up, g_per_tile)
    spmm_i = _make_spmm(ip, g_per_tile)

    def run_u(x4, v2):
        o = spmm_u(x4, src_u, dst_u, v2)
        return (o[:, 0] + o[:, 1])[:, :n_users]

    def run_i(x4, v2):
        o = spmm_i(x4, src_i, dst_i, v2)
        return (o[:, 0] + o[:, 1])[:, :n_items]

    eu4 = _split(embed_user)
    ei4 = _split(embed_item)

    users_e = run_u(ei4, v_adj) + eu4
    items_e = run_i(eu4, v_adj) + ei4

    di = d_i[None, :, None]
    dj = d_j[None, :, None]

    g1u = run_u(items_e, v_ui) + users_e * di
    g1i = run_i(users_e, v_iu) + items_e * dj
    g2u = run_u(g1i, v_ui) + g1u * di
    g2i = run_i(g1u, v_iu) + g1i * dj
    g3u = run_u(g2i, v_ui) + g2u * di
    g3i = run_i(g2u, v_iu) + g2i * dj
    g4u = run_u(g3i, v_ui) + g3u * di
    g4i = run_i(g3u, v_iu) + g3i * dj

    gcn_users = users_e + g1u * (1 / 2) + g2u * (1 / 3) + g3u * (1 / 4) + g4u
    gcn_items = items_e + g1i * (1 / 2) + g2i * (1 / 3) + g3i * (1 / 4) + g4i
    return (_unsplit(gcn_users), _unsplit(gcn_items))


# speedup vs baseline: 1.7473x; 1.7473x over previous
import functools

import jax
import jax.numpy as jnp
from jax import lax
from jax.experimental import pallas as pl
from jax.experimental.pallas import tpu as pltpu
from jax.experimental.pallas import tpu_sc as plsc

F = 64
FC = 16
P = F // FC
NC = 2
NS = 16
B = 128


def _spmm_body(n_dst_pad, g_per_tile, x_hbm, sidx_hbm, didx_hbm, val_hbm,
               out_hbm, sidx_v, didx_v, val_v, rows_v, zbuf_v, acc, gsem):
    c = lax.axis_index("c")
    s = lax.axis_index("s")
    w = c * NS + s
    zrows = n_dst_pad // NS

    def _zb(r, carry):
        zbuf_v[r] = jnp.zeros((FC,), jnp.float32)
        return carry

    lax.fori_loop(0, B, _zb, 0)

    for p in range(P):
        def _zero(i, carry):
            pltpu.sync_copy(zbuf_v, acc.at[pl.ds(s * zrows + i * B, B)])
            return carry

        lax.fori_loop(0, zrows // B, _zero, 0)
        plsc.subcore_barrier()

        def _batch(g, carry):
            row = w * g_per_tile + g
            pltpu.sync_copy(sidx_hbm.at[pl.ds(row * B, B)], sidx_v)
            pltpu.sync_copy(didx_hbm.at[pl.ds(row, 1)], didx_v)
            pltpu.sync_copy(val_hbm.at[pl.ds(row * B, B)], val_v)
            pltpu.async_copy(x_hbm.at[p].at[sidx_v], rows_v, gsem).wait()

            def _scale_grp(gg, carry):
                vv = val_v[pl.ds(gg * FC, FC)]

                def _scale(j, inner):
                    bc = vv.at[jnp.full((FC,), j, jnp.int32)].get(
                        mode='promise_in_bounds')
                    rows_v[gg * FC + j] = rows_v[gg * FC + j] * bc
                    return inner

                lax.fori_loop(0, FC, _scale, 0)
                return carry

            lax.fori_loop(0, B // FC, _scale_grp, 0)
            pltpu.sync_copy(rows_v, acc.at[didx_v.at[0]], add=True)
            return carry

        lax.fori_loop(0, g_per_tile, _batch, 0)
        plsc.subcore_barrier()

        pltpu.sync_copy(acc.at[pl.ds(s * zrows, zrows)],
                        out_hbm.at[p, c, pl.ds(s * zrows, zrows)])
        plsc.subcore_barrier()


def _make_spmm(n_dst_pad, g_per_tile):
    mesh = plsc.VectorSubcoreMesh(core_axis_name="c", subcore_axis_name="s",
                                  num_cores=NC, num_subcores=NS)
    return pl.kernel(
        functools.partial(_spmm_body, n_dst_pad, g_per_tile),
        out_type=jax.ShapeDtypeStruct((P, NC, n_dst_pad, FC), jnp.float32),
        mesh=mesh,
        compiler_params=pltpu.CompilerParams(use_tc_tiling_on_sc=False),
        scratch_types=[
            pltpu.VMEM((B,), jnp.int32),
            pltpu.VMEM((1, B), jnp.int32),
            pltpu.VMEM((B,), jnp.float32),
            pltpu.VMEM((B, FC), jnp.float32),
            pltpu.VMEM((B, FC), jnp.float32),
            pltpu.VMEM_SHARED((n_dst_pad, FC), jnp.float32),
            pltpu.SemaphoreType.DMA,
        ],
    )


def _ceil_to(x, m):
    return (x + m - 1) // m * m


def _split(x):
    n = x.shape[0]
    return x.reshape(n, P, FC).transpose(1, 0, 2)


def _unsplit(x4):
    n = x4.shape[1]
    return x4.transpose(1, 0, 2).reshape(n, F)


def kernel(embed_user, embed_item, u_idx, i_idx, ui_val, iu_val, adj_val, d_i, d_j):
    n_users, _ = embed_user.shape
    n_items = embed_item.shape[0]
    n_edges = u_idx.shape[0]

    e_pad = _ceil_to(n_edges, NC * NS * B)
    g_per_tile = e_pad // (NC * NS * B)
    up = _ceil_to(n_users, NS * B)
    ip = _ceil_to(n_items, NS * B)
    pad = e_pad - n_edges

    def prep_flat(idx):
        return jnp.pad(idx.astype(idx.dtype), (0, pad))

    def prep_idx(idx):
        return jnp.pad(idx.astype(jnp.int32), (0, pad)).reshape(-1, B)

    def prep_val(v):
        return jnp.pad(v, (0, pad)).reshape(-1, B)

    src_u = prep_flat(i_idx.astype(jnp.int32))
    dst_u = prep_idx(u_idx)
    src_i = prep_flat(u_idx.astype(jnp.int32))
    dst_i = prep_idx(i_idx)
    v_ui = prep_flat(ui_val)
    v_iu = prep_flat(iu_val)
    v_adj = prep_flat(adj_val)

    spmm_u = _make_spmm(up, g_per_tile)
    spmm_i = _make_spmm(ip, g_per_tile)

    def run_u(x4, v2):
        o = spmm_u(x4, src_u, dst_u, v2)
        return (o[:, 0] + o[:, 1])[:, :n_users]

    def run_i(x4, v2):
        o = spmm_i(x4, src_i, dst_i, v2)
        return (o[:, 0] + o[:, 1])[:, :n_items]

    eu4 = _split(embed_user)
    ei4 = _split(embed_item)

    users_e = run_u(ei4, v_adj) + eu4
    items_e = run_i(eu4, v_adj) + ei4

    di = d_i[None, :, None]
    dj = d_j[None, :, None]

    g1u = run_u(items_e, v_ui) + users_e * di
    g1i = run_i(users_e, v_iu) + items_e * dj
    g2u = run_u(g1i, v_ui) + g1u * di
    g2i = run_i(g1u, v_iu) + g1i * dj
    g3u = run_u(g2i, v_ui) + g2u * di
    g3i = run_i(g2u, v_iu) + g2i * dj
    g4u = run_u(g3i, v_ui) + g3u * di
    g4i = run_i(g3u, v_iu) + g3i * dj

    gcn_users = users_e + g1u * (1 / 2) + g2u * (1 / 3) + g3u * (1 / 4) + g4u
    gcn_items = items_e + g1i * (1 / 2) + g2i * (1 / 3) + g3i * (1 / 4) + g4i
    return (_unsplit(gcn_users), _unsplit(gcn_items))


# speedup vs baseline: 3.9234x; 2.2454x over previous
import functools

import jax
import jax.numpy as jnp
from jax import lax
from jax.experimental import pallas as pl
from jax.experimental.pallas import tpu as pltpu
from jax.experimental.pallas import tpu_sc as plsc

F = 64
FC = 16
P = F // FC
NC = 2
NS = 16
B = 128
SB = 1024
KG = SB // B


def _spmm_body(n_dst_pad, g_per_tile, x_hbm, sidx_hbm, didx_hbm, val_hbm,
               out_hbm, sidx_v, didx_v, val_v, rows_v, zbuf_v, acc, gsem):
    c = lax.axis_index("c")
    s = lax.axis_index("s")
    w = c * NS + s
    zrows = n_dst_pad // NS

    def _zb(r, carry):
        zbuf_v[r] = jnp.zeros((FC,), jnp.float32)
        return carry

    lax.fori_loop(0, B, _zb, 0)

    for p in range(P):
        def _zero(i, carry):
            pltpu.sync_copy(zbuf_v, acc.at[pl.ds(s * zrows + i * B, B)])
            return carry

        lax.fori_loop(0, zrows // B, _zero, 0)
        plsc.subcore_barrier()

        def _batch(g, carry):
            row = w * g_per_tile + g
            pltpu.sync_copy(sidx_hbm.at[pl.ds(row * SB, SB)], sidx_v)
            pltpu.sync_copy(didx_hbm.at[pl.ds(row * KG, KG)], didx_v)
            pltpu.sync_copy(val_hbm.at[pl.ds(row * SB, SB)], val_v)
            descs = [
                pltpu.async_copy(x_hbm.at[p].at[sidx_v.at[pl.ds(k * B, B)]],
                                 rows_v.at[pl.ds(k * B, B)], gsem)
                for k in range(KG)
            ]
            for d in descs:
                d.wait()

            def _scale_grp(gg, carry):
                vv = val_v[pl.ds(gg * FC, FC)]
                for j in range(FC):
                    bc = vv.at[jnp.full((FC,), j, jnp.int32)].get(
                        mode='promise_in_bounds')
                    rows_v[gg * FC + j] = rows_v[gg * FC + j] * bc
                return carry

            lax.fori_loop(0, SB // FC, _scale_grp, 0)
            for k in range(KG):
                pltpu.sync_copy(rows_v.at[pl.ds(k * B, B)],
                                acc.at[didx_v.at[k]], add=True)
            return carry

        lax.fori_loop(0, g_per_tile, _batch, 0)
        plsc.subcore_barrier()

        pltpu.sync_copy(acc.at[pl.ds(s * zrows, zrows)],
                        out_hbm.at[p, c, pl.ds(s * zrows, zrows)])
        plsc.subcore_barrier()


def _make_spmm(n_dst_pad, g_per_tile):
    mesh = plsc.VectorSubcoreMesh(core_axis_name="c", subcore_axis_name="s",
                                  num_cores=NC, num_subcores=NS)
    return pl.kernel(
        functools.partial(_spmm_body, n_dst_pad, g_per_tile),
        out_type=jax.ShapeDtypeStruct((P, NC, n_dst_pad, FC), jnp.float32),
        mesh=mesh,
        compiler_params=pltpu.CompilerParams(use_tc_tiling_on_sc=False),
        scratch_types=[
            pltpu.VMEM((SB,), jnp.int32),
            pltpu.VMEM((KG, B), jnp.int32),
            pltpu.VMEM((SB,), jnp.float32),
            pltpu.VMEM((SB, FC), jnp.float32),
            pltpu.VMEM((B, FC), jnp.float32),
            pltpu.VMEM_SHARED((n_dst_pad, FC), jnp.float32),
            pltpu.SemaphoreType.DMA,
        ],
    )


def _ceil_to(x, m):
    return (x + m - 1) // m * m


def _split(x):
    n = x.shape[0]
    return x.reshape(n, P, FC).transpose(1, 0, 2)


def _unsplit(x4):
    n = x4.shape[1]
    return x4.transpose(1, 0, 2).reshape(n, F)


def kernel(embed_user, embed_item, u_idx, i_idx, ui_val, iu_val, adj_val, d_i, d_j):
    n_users, _ = embed_user.shape
    n_items = embed_item.shape[0]
    n_edges = u_idx.shape[0]

    e_pad = _ceil_to(n_edges, NC * NS * SB)
    g_per_tile = e_pad // (NC * NS * SB)
    up = _ceil_to(n_users, NS * B)
    ip = _ceil_to(n_items, NS * B)
    pad = e_pad - n_edges

    def prep_flat(idx):
        return jnp.pad(idx.astype(idx.dtype), (0, pad))

    def prep_idx(idx):
        return jnp.pad(idx.astype(jnp.int32), (0, pad)).reshape(-1, B)

    def prep_val(v):
        return jnp.pad(v, (0, pad)).reshape(-1, B)

    src_u = prep_flat(i_idx.astype(jnp.int32))
    dst_u = prep_idx(u_idx)
    src_i = prep_flat(u_idx.astype(jnp.int32))
    dst_i = prep_idx(i_idx)
    v_ui = prep_flat(ui_val)
    v_iu = prep_flat(iu_val)
    v_adj = prep_flat(adj_val)

    spmm_u = _make_spmm(up, g_per_tile)
    spmm_i = _make_spmm(ip, g_per_tile)

    def run_u(x4, v2):
        o = spmm_u(x4, src_u, dst_u, v2)
        return (o[:, 0] + o[:, 1])[:, :n_users]

    def run_i(x4, v2):
        o = spmm_i(x4, src_i, dst_i, v2)
        return (o[:, 0] + o[:, 1])[:, :n_items]

    eu4 = _split(embed_user)
    ei4 = _split(embed_item)

    users_e = run_u(ei4, v_adj) + eu4
    items_e = run_i(eu4, v_adj) + ei4

    di = d_i[None, :, None]
    dj = d_j[None, :, None]

    g1u = run_u(items_e, v_ui) + users_e * di
    g1i = run_i(users_e, v_iu) + items_e * dj
    g2u = run_u(g1i, v_ui) + g1u * di
    g2i = run_i(g1u, v_iu) + g1i * dj
    g3u = run_u(g2i, v_ui) + g2u * di
    g3i = run_i(g2u, v_iu) + g2i * dj
    g4u = run_u(g3i, v_ui) + g3u * di
    g4i = run_i(g3u, v_iu) + g3i * dj

    gcn_users = users_e + g1u * (1 / 2) + g2u * (1 / 3) + g3u * (1 / 4) + g4u
    gcn_items = items_e + g1i * (1 / 2) + g2i * (1 / 3) + g3i * (1 / 4) + g4i
    return (_unsplit(gcn_users), _unsplit(gcn_items))


# speedup vs baseline: 4.2526x; 1.0839x over previous
import functools

import jax
import jax.numpy as jnp
from jax import lax
from jax.experimental import pallas as pl
from jax.experimental.pallas import tpu as pltpu
from jax.experimental.pallas import tpu_sc as plsc

F = 64
NC = 2
NS = 16
B = 128
VR = 16


NBUF = 3


def _spmm_body(fc, sb, n_dst_pad, g_per_tile, x_hbm, sidx_hbm, didx_hbm,
               val_hbm, out_hbm, sidx_v, didx_v, val_v, rows_v, zbuf_v, acc,
               ssem, gsem, csem):
    c = lax.axis_index("c")
    s = lax.axis_index("s")
    w = c * NS + s
    zrows = n_dst_pad // NS
    G = g_per_tile
    KG = sb // B
    P = F // fc
    H = fc // VR

    def _zb(r, carry):
        zbuf_v[r] = jnp.zeros((fc,), jnp.float32)
        return carry

    lax.fori_loop(0, B, _zb, 0)

    for p in range(P):
        def _zero(i, carry):
            pltpu.sync_copy(zbuf_v, acc.at[pl.ds(s * zrows + i * B, B)])
            return carry

        lax.fori_loop(0, zrows // B, _zero, 0)
        plsc.subcore_barrier()

        def _stage(g, b):
            row = w * G + g
            pltpu.async_copy(sidx_hbm.at[pl.ds(row * sb, sb)],
                             sidx_v.at[b], ssem.at[b])
            pltpu.async_copy(didx_hbm.at[pl.ds(row * KG, KG)],
                             didx_v.at[b], ssem.at[b])
            pltpu.async_copy(val_hbm.at[pl.ds(row * sb, sb)],
                             val_v.at[b], ssem.at[b])

        def _stage_wait(b):
            pltpu.make_async_copy(sidx_hbm.at[pl.ds(0, sb)],
                                  sidx_v.at[b], ssem.at[b]).wait()
            pltpu.make_async_copy(didx_hbm.at[pl.ds(0, KG)],
                                  didx_v.at[b], ssem.at[b]).wait()
            pltpu.make_async_copy(val_hbm.at[pl.ds(0, sb)],
                                  val_v.at[b], ssem.at[b]).wait()

        def _gather(b):
            for k in range(KG):
                pltpu.async_copy(
                    x_hbm.at[p].at[sidx_v.at[b, pl.ds(k * B, B)]],
                    rows_v.at[b, pl.ds(k * B, B)], gsem.at[b])

        def _gather_wait(b):
            for k in range(KG):
                pltpu.make_async_copy(
                    x_hbm.at[p].at[sidx_v.at[b, pl.ds(k * B, B)]],
                    rows_v.at[b, pl.ds(k * B, B)], gsem.at[b]).wait()

        def _scatter(b):
            for k in range(KG):
                pltpu.async_copy(rows_v.at[b, pl.ds(k * B, B)],
                                 acc.at[didx_v.at[b, k]], csem.at[b],
                                 add=True)

        def _scatter_wait(b):
            for k in range(KG):
                pltpu.make_async_copy(rows_v.at[b, pl.ds(k * B, B)],
                                      acc.at[didx_v.at[b, k]],
                                      csem.at[b]).wait()

        _stage(0, 0)
        _stage_wait(0)
        _gather(0)

        if G > 1:
            _stage(1, 1)

        def _step(g, carry):
            b = lax.rem(g, NBUF)
            bn = lax.rem(g + 1, NBUF)
            bs = lax.rem(g + 2, NBUF)

            @pl.when(g + 1 < G)
            def _():
                _stage_wait(bn)

            @pl.when(g >= 1)
            def _():
                _scatter_wait(bs)

            @pl.when(g + 2 < G)
            def _():
                _stage(g + 2, bs)

            _gather_wait(b)

            @pl.when(g + 1 < G)
            def _():
                _gather(bn)

            def _scale_grp(gg, inner):
                vv = val_v[b, pl.ds(gg * VR, VR)]
                for j in range(VR):
                    bc = vv.at[jnp.full((fc,), j, jnp.int32)].get(
                        mode='promise_in_bounds')
                    r = gg * VR + j
                    rows_v[b, r] = rows_v[b, r] * bc
                return inner

            lax.fori_loop(0, sb // VR, _scale_grp, 0)
            _scatter(b)
            return carry

        lax.fori_loop(0, G, _step, 0)
        _scatter_wait((G - 1) % NBUF)
        plsc.subcore_barrier()

        pltpu.sync_copy(acc.at[pl.ds(s * zrows, zrows)],
                        out_hbm.at[p, c, pl.ds(s * zrows, zrows)])
        plsc.subcore_barrier()


def _make_spmm(fc, sb, n_dst_pad, g_per_tile):
    mesh = plsc.VectorSubcoreMesh(core_axis_name="c", subcore_axis_name="s",
                                  num_cores=NC, num_subcores=NS)
    kg = sb // B
    return pl.kernel(
        functools.partial(_spmm_body, fc, sb, n_dst_pad, g_per_tile),
        out_type=jax.ShapeDtypeStruct((F // fc, NC, n_dst_pad, fc),
                                      jnp.float32),
        mesh=mesh,
        compiler_params=pltpu.CompilerParams(use_tc_tiling_on_sc=False),
        scratch_types=[
            pltpu.VMEM((NBUF, sb), jnp.int32),
            pltpu.VMEM((NBUF, kg, B), jnp.int32),
            pltpu.VMEM((NBUF, sb), jnp.float32),
            pltpu.VMEM((NBUF, sb, fc), jnp.float32),
            pltpu.VMEM((B, fc), jnp.float32),
            pltpu.VMEM_SHARED((n_dst_pad, fc), jnp.float32),
            pltpu.SemaphoreType.DMA((NBUF,)),
            pltpu.SemaphoreType.DMA((NBUF,)),
            pltpu.SemaphoreType.DMA((NBUF,)),
        ],
    )


def _ceil_to(x, m):
    return (x + m - 1) // m * m


def _split(x, fc):
    n = x.shape[0]
    return x.reshape(n, F // fc, fc).transpose(1, 0, 2)


def _to32(x4):
    n = x4.shape[1]
    return x4.reshape(2, 2, n, 16).transpose(0, 2, 1, 3).reshape(2, n, 32)


def _to16(x2):
    n = x2.shape[1]
    return x2.reshape(2, n, 2, 16).transpose(0, 2, 1, 3).reshape(4, n, 16)


def _unsplit16(x4):
    n = x4.shape[1]
    return x4.transpose(1, 0, 2).reshape(n, F)


SB_U = 128
SB_I = 512


def kernel(embed_user, embed_item, u_idx, i_idx, ui_val, iu_val, adj_val, d_i, d_j):
    n_users, _ = embed_user.shape
    n_items = embed_item.shape[0]
    n_edges = u_idx.shape[0]

    up = _ceil_to(n_users, NS * B)
    ip = _ceil_to(n_items, NS * B)

    def prep(sb, src, dst, vals):
        e_pad = _ceil_to(n_edges, NC * NS * sb)
        g_per_tile = e_pad // (NC * NS * sb)
        pad = e_pad - n_edges
        src_p = jnp.pad(src.astype(jnp.int32), (0, pad))
        dst_p = jnp.pad(dst.astype(jnp.int32), (0, pad)).reshape(-1, B)
        vals_p = [jnp.pad(v, (0, pad)) for v in vals]
        return g_per_tile, src_p, dst_p, vals_p

    gu, src_u, dst_u, (vu_ui, vu_adj) = prep(
        SB_U, i_idx, u_idx, [ui_val, adj_val])
    gi, src_i, dst_i, (vi_iu, vi_adj) = prep(
        SB_I, u_idx, i_idx, [iu_val, adj_val])

    spmm_u = _make_spmm(32, SB_U, up, gu)
    spmm_i = _make_spmm(16, SB_I, ip, gi)

    def run_u(x2, v2):
        o = spmm_u(x2, src_u, dst_u, v2)
        return (o[:, 0] + o[:, 1])[:, :n_users]

    def run_i(x4, v2):
        o = spmm_i(x4, src_i, dst_i, v2)
        return (o[:, 0] + o[:, 1])[:, :n_items]

    eu4 = _split(embed_user, 16)
    ei2 = _split(embed_item, 32)

    users_e = _to16(run_u(ei2, vu_adj)) + eu4
    items_e = run_i(eu4, vi_adj) + _to16(ei2)

    di = d_i[None, :, None]
    dj = d_j[None, :, None]

    def layer(prev_u4, prev_i4, term_u4, term_i4):
        gu_ = _to16(run_u(_to32(prev_i4), vu_ui)) + term_u4 * di
        gi_ = run_i(prev_u4, vi_iu) + term_i4 * dj
        return gu_, gi_

    g1u, g1i = layer(users_e, items_e, users_e, items_e)
    g2u, g2i = layer(g1u, g1i, g1u, g1i)
    g3u, g3i = layer(g2u, g2i, g2u, g2i)
    g4u, g4i = layer(g3u, g3i, g3u, g3i)

    gcn_users = users_e + g1u * (1 / 2) + g2u * (1 / 3) + g3u * (1 / 4) + g4u
    gcn_items = items_e + g1i * (1 / 2) + g2i * (1 / 3) + g3i * (1 / 4) + g4i
    return (_unsplit16(gcn_users), _unsplit16(gcn_items))
